# baseline (speedup 1.0000x reference)
import jax
import jax.numpy as jnp
from jax.experimental import pallas as pl

_NUM_EXPERTS = 16
_TOP_K = 2
_BLOCK_ROWS = 128


def _select_top2(logits_t, idx_ref, val_ref):
    row = jax.lax.broadcasted_iota(jnp.int32, logits_t.shape, 0)
    revf = (15 - row).astype(jnp.float32)

    m1 = jnp.max(logits_t, axis=0, keepdims=True)
    r1 = jnp.max(jnp.where(logits_t == m1, revf, -1.0), axis=0, keepdims=True)

    masked = jnp.where(revf == r1, -jnp.inf, logits_t)
    m2 = jnp.max(masked, axis=0, keepdims=True)
    r2 = jnp.max(jnp.where(masked == m2, revf, -1.0), axis=0, keepdims=True)

    ex = jnp.exp(logits_t - m1)
    denom = jnp.sum(ex, axis=0, keepdims=True)
    v1 = 1.0 / denom
    v2 = jnp.exp(m2 - m1) / denom

    i1 = (15.0 - r1).astype(jnp.int32)
    i2 = (15.0 - r2).astype(jnp.int32)
    idx_ref[...] = jnp.concatenate([i1, i2], axis=0)
    val_ref[...] = jnp.concatenate([v1, v2], axis=0)


def _gate_kernel(x1_ref, x2_ref, w_ref, idx1_ref, val1_ref, idx2_ref, val2_ref):
    w = w_ref[...]
    dims = (((1,), (1,)), ((), ()))
    lt1 = jax.lax.dot_general(
        w, x1_ref[...], dims, preferred_element_type=jnp.float32
    )
    _select_top2(lt1, idx1_ref, val1_ref)
    lt2 = jax.lax.dot_general(
        w, x2_ref[...], dims, preferred_element_type=jnp.float32
    )
    _select_top2(lt2, idx2_ref, val2_ref)


@jax.jit
def kernel(hidden_states, weight):
    d = hidden_states.shape[-1]
    hs = hidden_states.reshape(-1, d)
    t = hs.shape[0]
    half = t // 2
    nblk = half // _BLOCK_ROWS
    grid = (nblk,)

    idx1, val1, idx2, val2 = pl.pallas_call(
        _gate_kernel,
        grid=grid,
        in_specs=[
            pl.BlockSpec((_BLOCK_ROWS, d), lambda i: (i, 0)),
            pl.BlockSpec((_BLOCK_ROWS, d), lambda i, nb=nblk: (i + nb, 0)),
            pl.BlockSpec((_NUM_EXPERTS, d), lambda i: (0, 0)),
        ],
        out_specs=[
            pl.BlockSpec((_TOP_K, _BLOCK_ROWS), lambda i: (0, i)),
            pl.BlockSpec((_TOP_K, _BLOCK_ROWS), lambda i: (0, i)),
            pl.BlockSpec((_TOP_K, _BLOCK_ROWS), lambda i: (0, i)),
            pl.BlockSpec((_TOP_K, _BLOCK_ROWS), lambda i: (0, i)),
        ],
        out_shape=[
            jax.ShapeDtypeStruct((_TOP_K, half), jnp.int32),
            jax.ShapeDtypeStruct((_TOP_K, half), jnp.float32),
            jax.ShapeDtypeStruct((_TOP_K, half), jnp.int32),
            jax.ShapeDtypeStruct((_TOP_K, half), jnp.float32),
        ],
    )(hs, hs, weight)
    idx = jnp.concatenate([idx1, idx2], axis=1).T
    val = jnp.concatenate([val1, val2], axis=1).T
    return idx, val


# speedup vs baseline: 1.4782x; 1.4782x over previous
import jax
import jax.numpy as jnp
from jax.experimental import pallas as pl

_NUM_EXPERTS = 16
_TOP_K = 2
_BLOCK_ROWS = 512


def _select_top2(logits_t, idx_ref, val_ref):
    row = jax.lax.broadcasted_iota(jnp.int32, logits_t.shape, 0)
    revf = (15 - row).astype(jnp.float32)

    m1 = jnp.max(logits_t, axis=0, keepdims=True)
    r1 = jnp.max(jnp.where(logits_t == m1, revf, -1.0), axis=0, keepdims=True)

    masked = jnp.where(revf == r1, -jnp.inf, logits_t)
    m2 = jnp.max(masked, axis=0, keepdims=True)
    r2 = jnp.max(jnp.where(masked == m2, revf, -1.0), axis=0, keepdims=True)

    ex = jnp.exp(logits_t - m1)
    denom = jnp.sum(ex, axis=0, keepdims=True)
    v1 = 1.0 / denom
    v2 = jnp.exp(m2 - m1) / denom

    i1 = (15.0 - r1).astype(jnp.int32)
    i2 = (15.0 - r2).astype(jnp.int32)
    idx_ref[...] = jnp.concatenate([i1, i2], axis=0)
    val_ref[...] = jnp.concatenate([v1, v2], axis=0)


def _gate_kernel(x1_ref, x2_ref, w_ref, idx1_ref, val1_ref, idx2_ref, val2_ref):
    w = w_ref[...]
    dims = (((1,), (1,)), ((), ()))
    lt1 = jax.lax.dot_general(
        w, x1_ref[...], dims, preferred_element_type=jnp.float32
    )
    _select_top2(lt1, idx1_ref, val1_ref)
    lt2 = jax.lax.dot_general(
        w, x2_ref[...], dims, preferred_element_type=jnp.float32
    )
    _select_top2(lt2, idx2_ref, val2_ref)


@jax.jit
def kernel(hidden_states, weight):
    d = hidden_states.shape[-1]
    hs = hidden_states.reshape(-1, d)
    t = hs.shape[0]
    half = t // 2
    nblk = half // _BLOCK_ROWS
    grid = (nblk,)

    idx1, val1, idx2, val2 = pl.pallas_call(
        _gate_kernel,
        grid=grid,
        in_specs=[
            pl.BlockSpec((_BLOCK_ROWS, d), lambda i: (i, 0)),
            pl.BlockSpec((_BLOCK_ROWS, d), lambda i, nb=nblk: (i + nb, 0)),
            pl.BlockSpec((_NUM_EXPERTS, d), lambda i: (0, 0)),
        ],
        out_specs=[
            pl.BlockSpec((_TOP_K, _BLOCK_ROWS), lambda i: (0, i)),
            pl.BlockSpec((_TOP_K, _BLOCK_ROWS), lambda i: (0, i)),
            pl.BlockSpec((_TOP_K, _BLOCK_ROWS), lambda i: (0, i)),
            pl.BlockSpec((_TOP_K, _BLOCK_ROWS), lambda i: (0, i)),
        ],
        out_shape=[
            jax.ShapeDtypeStruct((_TOP_K, half), jnp.int32),
            jax.ShapeDtypeStruct((_TOP_K, half), jnp.float32),
            jax.ShapeDtypeStruct((_TOP_K, half), jnp.int32),
            jax.ShapeDtypeStruct((_TOP_K, half), jnp.float32),
        ],
    )(hs, hs, weight)
    idx = jnp.concatenate([idx1, idx2], axis=1).T
    val = jnp.concatenate([val1, val2], axis=1).T
    return idx, val
